# baseline (speedup 1.0000x reference)
import functools

import jax
import jax.numpy as jnp
from jax import lax
from jax.experimental import pallas as pl
from jax.experimental.pallas import tpu as pltpu
from jax.experimental.pallas import tpu_sc as plsc

N = 10000
E = 160000
IN_W = 128
WIDTH = 16
DEPTH = 4
HPAD = 128

NUM_CORES = 2
NUM_SUBCORES = 16
NUM_WORKERS = NUM_CORES * NUM_SUBCORES
CHUNK = 128
NCHUNKS = E // CHUNK
CHUNKS_PER_WORKER = -(-NCHUNKS // NUM_WORKERS)
CHUNKS_PER_TILE = -(-NCHUNKS // NUM_SUBCORES)
HALF = N // NUM_CORES

_f32 = jnp.float32


def _pad_cols(h):
    return jnp.concatenate(
        [h, jnp.zeros((h.shape[0], HPAD - WIDTH), _f32)], axis=1)


def _lift_body(x_ref, wff1_ref, bff1_ref, wff2_ref, bff2_ref, wfc1_ref,
               bfc1_ref, o_ref):
    x = x_ref[...]
    h = jnp.sin(jnp.dot(x, wff1_ref[...], preferred_element_type=_f32)
                + bff1_ref[...])
    h = (jnp.dot(h, wff2_ref[...], preferred_element_type=_f32) + bff2_ref[...]
         + jnp.dot(x, wfc1_ref[...], preferred_element_type=_f32)
         + bfc1_ref[...])
    o_ref[...] = _pad_cols(h)


def _lift(x, Wff1, bff1, Wff2, bff2, Wfc1, bfc1):
    return pl.pallas_call(
        _lift_body,
        out_shape=jax.ShapeDtypeStruct((N, HPAD), _f32),
    )(x, Wff1, bff1.reshape(1, WIDTH), Wff2, bff2.reshape(1, WIDTH),
      Wfc1, bfc1.reshape(1, WIDTH))


EBLK = 1600


def _msg_body(ea_ref, hs_ref, wk1_ref, bk1_ref, wk2_ref, bk2_ref, wk3_ref,
              bk3_ref, o_ref):
    e = jax.nn.relu(jnp.dot(ea_ref[...], wk1_ref[...],
                            preferred_element_type=_f32) + bk1_ref[...])
    e = jax.nn.relu(jnp.dot(e, wk2_ref[...],
                            preferred_element_type=_f32) + bk2_ref[...])
    w = (jnp.dot(e, wk3_ref[...], preferred_element_type=_f32)
         + bk3_ref[...])
    hs = hs_ref[...]
    m = hs[:, 0:1] * w[:, :WIDTH]
    for i in range(1, WIDTH):
        m = m + hs[:, i:i + 1] * w[:, i * WIDTH:(i + 1) * WIDTH]
    o_ref[...] = m


def _msg(edge_attr, hsrc, Wk1, bk1, Wk2, bk2, Wk3, bk3):
    ki, h1 = Wk1.shape
    h2 = Wk2.shape[1]
    w2 = Wk3.shape[1]
    return pl.pallas_call(
        _msg_body,
        grid=(E // EBLK,),
        in_specs=[
            pl.BlockSpec((EBLK, ki), lambda i: (i, 0)),
            pl.BlockSpec((EBLK, WIDTH), lambda i: (i, 0)),
            pl.BlockSpec((ki, h1), lambda i: (0, 0)),
            pl.BlockSpec((1, h1), lambda i: (0, 0)),
            pl.BlockSpec((h1, h2), lambda i: (0, 0)),
            pl.BlockSpec((1, h2), lambda i: (0, 0)),
            pl.BlockSpec((h2, w2), lambda i: (0, 0)),
            pl.BlockSpec((1, w2), lambda i: (0, 0)),
        ],
        out_specs=pl.BlockSpec((EBLK, WIDTH), lambda i: (i, 0)),
        out_shape=jax.ShapeDtypeStruct((E, WIDTH), _f32),
    )(edge_attr, hsrc, Wk1, bk1.reshape(1, h1), Wk2, bk2.reshape(1, h2),
      Wk3, bk3.reshape(1, w2))


def _reduce_body(p_ref, o_ref, *, clamp_one):
    s = p_ref[pl.ds(0, 1), :]
    for t in range(1, NUM_SUBCORES):
        s = s + p_ref[pl.ds(t, 1), :]
    if clamp_one:
        s = jnp.maximum(s, 1.0)
    o_ref[...] = s


def _reduce_partials(raw, clamp_one):
    cols = NUM_CORES * ACC_DATA_ROWS * HPAD
    flat = raw.reshape(NUM_SUBCORES, cols)
    s = pl.pallas_call(
        functools.partial(_reduce_body, clamp_one=clamp_one),
        out_shape=jax.ShapeDtypeStruct((1, cols), _f32),
    )(flat)
    p = s.reshape(NUM_CORES, ACC_DATA_ROWS * 8, WIDTH)
    return jnp.concatenate([p[0, :HALF], p[1, :HALF]], axis=0)


def _update_body(agg_ref, cnt_ref, h_ref, root_ref, cbias_ref, o_ref, *,
                 relu):
    h = (agg_ref[...] / cnt_ref[...]
         + jnp.dot(h_ref[:, :WIDTH], root_ref[...],
                   preferred_element_type=_f32)
         + cbias_ref[...])
    if relu:
        h = jax.nn.relu(h)
    o_ref[...] = _pad_cols(h)


def _update(agg, cnt, h, root, conv_bias, relu):
    return pl.pallas_call(
        functools.partial(_update_body, relu=relu),
        out_shape=jax.ShapeDtypeStruct((N, HPAD), _f32),
    )(agg, cnt, h, root, conv_bias.reshape(1, WIDTH))


def _update_final_body(agg_ref, cnt_ref, h_ref, root_ref, cbias_ref,
                       wfc2_ref, bfc2_ref, o_ref):
    h = (agg_ref[...] / cnt_ref[...]
         + jnp.dot(h_ref[:, :WIDTH], root_ref[...],
                   preferred_element_type=_f32)
         + cbias_ref[...])
    o_ref[...] = (jnp.dot(h, wfc2_ref[...], preferred_element_type=_f32)
                  + bfc2_ref[...])


def _update_final(agg, cnt, h, root, conv_bias, Wfc2, bfc2):
    return pl.pallas_call(
        _update_final_body,
        out_shape=jax.ShapeDtypeStruct((N, 1), _f32),
    )(agg, cnt, h, root, conv_bias.reshape(1, WIDTH), Wfc2,
      bfc2.reshape(1, 1))


_MESH = plsc.VectorSubcoreMesh(core_axis_name="core", subcore_axis_name="subcore")


def _sc_gather_body(h_hbm, src_hbm, out_hbm, srcv, rows, packed, sem):
    cid = lax.axis_index("core")
    sid = lax.axis_index("subcore")
    wid = cid * NUM_SUBCORES + sid

    @pl.loop(0, CHUNKS_PER_WORKER)
    def _(i):
        c = wid + NUM_WORKERS * i

        @pl.when(c < NCHUNKS)
        def _():
            base = pl.multiple_of(c * CHUNK, 8)
            pltpu.sync_copy(src_hbm.at[pl.ds(base, CHUNK)], srcv)
            pltpu.async_copy(h_hbm.at[srcv], rows, sem).wait()
            for e in range(CHUNK):
                packed[e, :] = rows[e, 0:WIDTH]
            pltpu.sync_copy(packed, out_hbm.at[pl.ds(base, CHUNK)])


def _sc_gather(h, src):
    k = pl.kernel(
        _sc_gather_body,
        out_type=jax.ShapeDtypeStruct((E, WIDTH), _f32),
        mesh=_MESH,
        scratch_types=[
            pltpu.VMEM((CHUNK,), jnp.int32),
            pltpu.VMEM((CHUNK, HPAD), _f32),
            pltpu.VMEM((CHUNK, WIDTH), _f32),
            pltpu.SemaphoreType.DMA,
        ],
    )
    return k(h, src)


ACC_DATA_ROWS = 640
ACC_ROWS = 648
DUMP_NODE = ACC_DATA_ROWS * 8


def _scatter_edge(acc, dv, j, base_node, val):
    n = dv[j] - base_node
    ok = (n >= 0) & (n < HALF)
    nc = jnp.where(ok, n, DUMP_NODE)
    row = nc // 8
    off = (nc % 8) * WIDTH
    acc[row, pl.ds(off, WIDTH)] = acc[row, pl.ds(off, WIDTH)] + val


def _sc_scatter_body(msg_hbm, dst_hbm, zeros_hbm, out_hbm, dstv, msgv, acc):
    cid = lax.axis_index("core")
    sid = lax.axis_index("subcore")
    base_node = cid * HALF

    pltpu.sync_copy(zeros_hbm, acc)

    @pl.loop(0, CHUNKS_PER_TILE)
    def _(i):
        c = sid + NUM_SUBCORES * i

        @pl.when(c < NCHUNKS)
        def _():
            base = pl.multiple_of(c * CHUNK, 8)
            pltpu.sync_copy(dst_hbm.at[pl.ds(base, CHUNK)], dstv)
            pltpu.sync_copy(msg_hbm.at[pl.ds(base, CHUNK)], msgv)
            for g in range(CHUNK // 16):
                dv = dstv[pl.ds(g * 16, 16)]
                for j in range(16):
                    _scatter_edge(acc, dv, j, base_node, msgv[g * 16 + j, :])

    pltpu.sync_copy(acc.at[pl.ds(0, ACC_DATA_ROWS)],
                    out_hbm.at[pl.ds((sid * NUM_CORES + cid) * ACC_DATA_ROWS,
                                     ACC_DATA_ROWS)])


def _sc_scatter(msg, dst, zeros_acc):
    k = pl.kernel(
        _sc_scatter_body,
        out_type=jax.ShapeDtypeStruct(
            (NUM_WORKERS * ACC_DATA_ROWS, HPAD), _f32),
        mesh=_MESH,
        scratch_types=[
            pltpu.VMEM((CHUNK,), jnp.int32),
            pltpu.VMEM((CHUNK, WIDTH), _f32),
            pltpu.VMEM((ACC_ROWS, HPAD), _f32),
        ],
    )
    return k(msg, dst, zeros_acc)


def _sc_cnt_body(dst_hbm, zeros_hbm, out_hbm, dstv, acc):
    cid = lax.axis_index("core")
    sid = lax.axis_index("subcore")
    base_node = cid * HALF
    onev = jnp.ones((16,), _f32)

    pltpu.sync_copy(zeros_hbm, acc)

    @pl.loop(0, CHUNKS_PER_TILE)
    def _(i):
        c = sid + NUM_SUBCORES * i

        @pl.when(c < NCHUNKS)
        def _():
            base = pl.multiple_of(c * CHUNK, 8)
            pltpu.sync_copy(dst_hbm.at[pl.ds(base, CHUNK)], dstv)
            for g in range(CHUNK // 16):
                dv = dstv[pl.ds(g * 16, 16)]
                for j in range(16):
                    _scatter_edge(acc, dv, j, base_node, onev)

    pltpu.sync_copy(acc.at[pl.ds(0, ACC_DATA_ROWS)],
                    out_hbm.at[pl.ds((sid * NUM_CORES + cid) * ACC_DATA_ROWS,
                                     ACC_DATA_ROWS)])


def _sc_cnt(dst, zeros_acc):
    k = pl.kernel(
        _sc_cnt_body,
        out_type=jax.ShapeDtypeStruct(
            (NUM_WORKERS * ACC_DATA_ROWS, HPAD), _f32),
        mesh=_MESH,
        scratch_types=[
            pltpu.VMEM((CHUNK,), jnp.int32),
            pltpu.VMEM((ACC_ROWS, HPAD), _f32),
        ],
    )
    return k(dst, zeros_acc)


def kernel(x, edge_index, edge_attr, Wff1, bff1, Wff2, bff2, Wfc1, bfc1,
           Wk1, bk1, Wk2, bk2, Wk3, bk3, root, conv_bias, Wfc2, bfc2):
    src = edge_index[0].astype(jnp.int32)
    dst = edge_index[1].astype(jnp.int32)

    h = _lift(x, Wff1, bff1, Wff2, bff2, Wfc1, bfc1)

    zeros_acc = jnp.zeros((ACC_ROWS, HPAD), _f32)

    cnt = _reduce_partials(_sc_cnt(dst, zeros_acc), clamp_one=True)
    for k in range(DEPTH):
        hsrc = _sc_gather(h, src)
        msg = _msg(edge_attr, hsrc, Wk1, bk1, Wk2, bk2, Wk3, bk3)
        agg = _reduce_partials(_sc_scatter(msg, dst, zeros_acc),
                               clamp_one=False)
        if k < DEPTH - 1:
            h = _update(agg, cnt, h, root, conv_bias, relu=True)
    return _update_final(agg, cnt, h, root, conv_bias, Wfc2, bfc2)
